# Initial kernel scaffold; baseline (speedup 1.0000x reference)
import jax
import jax.numpy as jnp
from jax.experimental import pallas as pl


def kernel(x, ln1_g, ln1_b, w_qkv, b_qkv, w_proj, b_proj, ln2_g, ln2_b, w_fc1, b_fc1, w_fc2, b_fc2):
    raise NotImplementedError("write your pallas kernel here")


# speedup vs baseline: 1.3531x; 1.3531x over previous
import functools

import numpy as np

import jax
import jax.numpy as jnp
from jax.experimental import pallas as pl
from jax.experimental.pallas import tpu as pltpu

WS = 4
SS = 2
NH = 12
C = 384
HID = 1536
DEPTH = 2

_NEG = -1e9


def _build_masks(D, H, W):
    NT = 4 * 4 * W
    wid = np.broadcast_to((np.arange(W) // WS)[None, None, :], (4, 4, W)).reshape(NT)
    same_win = wid[:, None] == wid[None, :]
    mask1 = np.where(same_win, 0.0, _NEG).astype(np.float32)

    img = np.zeros((D, H, W), np.int32)
    cnt = 0
    sl = (slice(0, -WS), slice(-WS, -SS), slice(-SS, None))
    for d in sl:
        for h in sl:
            for w in sl:
                img[d, h, w] = cnt
                cnt += 1
    nDB, nHB = D // WS, H // WS
    masks2 = np.empty((nDB * nHB, NT, NT), np.float32)
    for dB in range(nDB):
        for hB in range(nHB):
            reg = img[4 * dB:4 * dB + 4, 4 * hB:4 * hB + 4, :].reshape(NT)
            diff = reg[:, None] != reg[None, :]
            masks2[dB * nHB + hB] = np.where(
                same_win, np.where(diff, -100.0, 0.0), _NEG)
    return mask1, masks2


def _bdot(a, b):
    return jax.lax.dot_general(
        a, b, (((1,), (0,)), ((), ())), preferred_element_type=jnp.float32)


def _block_body(x_ref, mask_ref, wqkv_ref, bqkv_ref, wproj_ref, bproj_ref,
                ln1g_ref, ln1b_ref, ln2g_ref, ln2b_ref,
                wfc1_ref, bfc1_ref, wfc2_ref, bfc2_ref, out_ref):
    bf16 = jnp.bfloat16
    blk = x_ref[0]
    d4, h4, W, Ch = blk.shape
    NT = d4 * h4 * W
    xt = blk.reshape(NT, Ch)

    mu = jnp.mean(xt, axis=-1, keepdims=True)
    var = jnp.mean((xt - mu) ** 2, axis=-1, keepdims=True)
    h = (xt - mu) / jnp.sqrt(var + 1e-5) * ln1g_ref[0] + ln1b_ref[0]

    qkv = _bdot(h.astype(bf16), wqkv_ref[...]) + bqkv_ref[0]
    hd = Ch // NH
    scale = hd ** -0.5
    q = (qkv[:, :Ch] * scale).astype(bf16)
    kT = jnp.swapaxes(qkv[:, Ch:2 * Ch].astype(bf16), 0, 1)
    v = qkv[:, 2 * Ch:].astype(bf16)
    maskf = mask_ref[0].astype(jnp.float32)

    outs = []
    for i in range(NH):
        s = _bdot(q[:, i * hd:(i + 1) * hd], kT[i * hd:(i + 1) * hd, :])
        p = jax.nn.softmax(s + maskf, axis=-1)
        outs.append(_bdot(p.astype(bf16), v[:, i * hd:(i + 1) * hd]))
    ao = jnp.concatenate(outs, axis=1)
    ao = _bdot(ao.astype(bf16), wproj_ref[...]) + bproj_ref[0]

    x2 = xt + ao

    mu2 = jnp.mean(x2, axis=-1, keepdims=True)
    var2 = jnp.mean((x2 - mu2) ** 2, axis=-1, keepdims=True)
    y = (x2 - mu2) / jnp.sqrt(var2 + 1e-5) * ln2g_ref[0] + ln2b_ref[0]
    y = _bdot(y.astype(bf16), wfc1_ref[...]) + bfc1_ref[0]
    y = jax.nn.gelu(y)
    y = _bdot(y.astype(bf16), wfc2_ref[...]) + bfc2_ref[0]

    out_ref[0] = (x2 + y).reshape(d4, h4, W, Ch)


def _roll_body(x_ref, out_ref, sh):
    out_ref[0] = jnp.roll(x_ref[0], (sh, sh, sh), axis=(0, 1, 2))


def _roll3(x, sh):
    B, D, H, W, Ch = x.shape
    cblk = 128
    return pl.pallas_call(
        functools.partial(_roll_body, sh=sh),
        grid=(B, Ch // cblk),
        in_specs=[pl.BlockSpec((1, D, H, W, cblk), lambda b, c: (b, 0, 0, 0, c))],
        out_specs=pl.BlockSpec((1, D, H, W, cblk), lambda b, c: (b, 0, 0, 0, c)),
        out_shape=jax.ShapeDtypeStruct(x.shape, x.dtype),
        compiler_params=pltpu.CompilerParams(
            dimension_semantics=("parallel", "parallel")),
    )(x)


def _swin_block(x, mask_arr, n_mask, params):
    B, D, H, W, Ch = x.shape
    nDB, nHB = D // WS, H // WS
    NT = 4 * 4 * W
    (wqkv, bqkv, wproj, bproj, ln1g, ln1b, ln2g, ln2b,
     wfc1, bfc1, wfc2, bfc2) = params

    if n_mask == 1:
        mask_imap = lambda b, d, h: (0, 0, 0)
    else:
        mask_imap = lambda b, d, h: (d * nHB + h, 0, 0)

    def _const(shape):
        nd = len(shape)
        return pl.BlockSpec(shape, lambda b, d, h, _nd=nd: (0,) * _nd)

    return pl.pallas_call(
        _block_body,
        grid=(B, nDB, nHB),
        in_specs=[
            pl.BlockSpec((1, 4, 4, W, Ch), lambda b, d, h: (b, d, h, 0, 0)),
            pl.BlockSpec((1, NT, NT), mask_imap),
            _const(wqkv.shape), _const(bqkv.shape),
            _const(wproj.shape), _const(bproj.shape),
            _const(ln1g.shape), _const(ln1b.shape),
            _const(ln2g.shape), _const(ln2b.shape),
            _const(wfc1.shape), _const(bfc1.shape),
            _const(wfc2.shape), _const(bfc2.shape),
        ],
        out_specs=pl.BlockSpec((1, 4, 4, W, Ch), lambda b, d, h: (b, d, h, 0, 0)),
        out_shape=jax.ShapeDtypeStruct(x.shape, jnp.float32),
        compiler_params=pltpu.CompilerParams(
            dimension_semantics=("parallel", "parallel", "parallel")),
    )(x, mask_arr, wqkv, bqkv, wproj, bproj, ln1g, ln1b, ln2g, ln2b,
      wfc1, bfc1, wfc2, bfc2)


def kernel(x, ln1_g, ln1_b, w_qkv, b_qkv, w_proj, b_proj,
           ln2_g, ln2_b, w_fc1, b_fc1, w_fc2, b_fc2):
    B, D, H, W, Ch = x.shape
    bf16 = jnp.bfloat16
    m1, m2 = _build_masks(D, H, W)
    mask1 = jnp.asarray(m1, dtype=bf16)[None]
    masks2 = jnp.asarray(m2, dtype=bf16)

    def params(i):
        return (w_qkv[i].astype(bf16), b_qkv[i][None],
                w_proj[i].astype(bf16), b_proj[i][None],
                ln1_g[i][None], ln1_b[i][None],
                ln2_g[i][None], ln2_b[i][None],
                w_fc1[i].astype(bf16), b_fc1[i][None],
                w_fc2[i].astype(bf16), b_fc2[i][None])

    x = _swin_block(x, mask1, 1, params(0))
    x = _roll3(x, -SS)
    x = _swin_block(x, masks2, masks2.shape[0], params(1))
    x = _roll3(x, SS)
    return x
